# Initial kernel scaffold; baseline (speedup 1.0000x reference)
import jax
import jax.numpy as jnp
from jax.experimental import pallas as pl


def kernel(x, edge_index, batch, params):
    raise NotImplementedError("write your pallas kernel here")


# speedup vs baseline: 3.5207x; 3.5207x over previous
import functools

import jax
import jax.numpy as jnp
from jax import lax
from jax.experimental import pallas as pl
from jax.experimental.pallas import tpu as pltpu
from jax.experimental.pallas import tpu_sc as plsc

N = 10000
E = 320000
D = 128
H = 128
C = 10
G = 64

NC = 2
NS = 16
NW = NC * NS
EPT = E // NW
K = 128
CH = 80
EPAD = CH * K
ROWS_PER_TILE = 632
AGG_ROWS = ROWS_PER_TILE * NS


def _sc_agg(h, pidx, zeros):
    mesh = plsc.VectorSubcoreMesh(core_axis_name="c", subcore_axis_name="s")

    @functools.partial(
        pl.kernel,
        mesh=mesh,
        out_type=jax.ShapeDtypeStruct((NC, AGG_ROWS, H), jnp.float32),
        scratch_types=[
            pltpu.VMEM((2, K), jnp.int32),
            pltpu.VMEM((2, K), jnp.int32),
            pltpu.VMEM((K, H), jnp.float32),
            pltpu.VMEM((K, H), jnp.float32),
            pltpu.VMEM_SHARED((AGG_ROWS, H), jnp.float32),
            pltpu.SemaphoreType.DMA,
            pltpu.SemaphoreType.DMA,
            pltpu.SemaphoreType.DMA,
            pltpu.SemaphoreType.DMA,
        ],
    )
    def k(h_hbm, pidx_hbm, z_hbm, out_hbm,
          ibuf0, ibuf1, rows0, rows1, agg_sh, isem0, isem1, gsem0, gsem1):
        c = lax.axis_index("c")
        s = lax.axis_index("s")
        wid = c * NS + s
        ibuf = (ibuf0, ibuf1)
        rows = (rows0, rows1)
        isem = (isem0, isem1)
        gsem = (gsem0, gsem1)

        pltpu.sync_copy(z_hbm.at[pl.ds(s * ROWS_PER_TILE, ROWS_PER_TILE)],
                        agg_sh.at[pl.ds(s * ROWS_PER_TILE, ROWS_PER_TILE)])
        plsc.subcore_barrier()

        def start_idx(j, b):
            pltpu.async_copy(pidx_hbm.at[wid, j], ibuf[b], isem[b])

        def wait_idx(b):
            pltpu.make_async_copy(pidx_hbm.at[0, 0], ibuf[b], isem[b]).wait()

        def start_gather(b):
            pltpu.async_copy(h_hbm.at[ibuf[b].at[0]], rows[b], gsem[b])

        def wait_gather(b):
            pltpu.make_async_copy(h_hbm.at[pl.ds(0, K)], rows[b],
                                  gsem[b]).wait()

        def scatter(b):
            pltpu.sync_copy(rows[b], agg_sh.at[ibuf[b].at[1]], add=True)

        start_idx(0, 0)
        wait_idx(0)
        start_gather(0)
        start_idx(1, 1)
        wait_idx(1)
        start_gather(1)
        wait_gather(0)
        scatter(0)
        start_idx(2, 0)

        def pair(jj, carry):
            for b in range(2):
                j = 2 * jj + 2 + b
                wait_idx(b)
                start_gather(b)
                wait_gather(1 - b)
                scatter(1 - b)
                pltpu.async_copy(
                    pidx_hbm.at[wid, jnp.minimum(j + 1, CH - 1)],
                    ibuf[1 - b], isem[1 - b])
            return carry

        lax.fori_loop(0, (CH - 2) // 2, pair, 0)
        wait_idx(0)
        wait_gather(1)
        scatter(1)

        plsc.subcore_barrier()
        pltpu.sync_copy(agg_sh.at[pl.ds(s * ROWS_PER_TILE, ROWS_PER_TILE)],
                        out_hbm.at[c, pl.ds(s * ROWS_PER_TILE, ROWS_PER_TILE)])

    return k(h, pidx, zeros)


def _tc_layer(h, agg2, scale, W1, b1, W2, b2, gamma, beta):
    def body(h_ref, agg_ref, sc_ref, w1_ref, b1_ref, w2_ref, b2_ref,
             g_ref, be_ref, out_ref):
        z = (h_ref[...] * sc_ref[0, 0] + agg_ref[0, pl.ds(0, N)]
             + agg_ref[1, pl.ds(0, N)])
        z = jnp.maximum(
            jnp.dot(z, w1_ref[...], preferred_element_type=jnp.float32) + b1_ref[...], 0.0)
        z = jnp.maximum(
            jnp.dot(z, w2_ref[...], preferred_element_type=jnp.float32) + b2_ref[...], 0.0)
        mean = jnp.sum(z, axis=0, keepdims=True) * (1.0 / N)
        d = z - mean
        var = jnp.sum(d * d, axis=0, keepdims=True) * (1.0 / N)
        zn = d * (g_ref[...] * lax.rsqrt(var + 1e-5)) + be_ref[...]
        out_ref[...] = jnp.maximum(zn, 0.0)

    return pl.pallas_call(
        body,
        out_shape=jax.ShapeDtypeStruct((N, H), jnp.float32),
    )(h, agg2, scale, W1, b1, W2, b2, gamma, beta)


def _tc_head(h, batch2d, fc1W, fc1b, fc2W, fc2b, fc3W, fc3b):
    def body(h_ref, b_ref, w1_ref, b1_ref, w2_ref, b2_ref, w3_ref, b3_ref,
             out_ref):
        seg_ids = lax.broadcasted_iota(jnp.int32, (G, N), 0)
        onehot = (seg_ids == jnp.broadcast_to(b_ref[...], (G, N))
                  ).astype(jnp.float32)
        pooled = jnp.dot(onehot, h_ref[...], preferred_element_type=jnp.float32)
        z = jnp.maximum(
            jnp.dot(pooled, w1_ref[...], preferred_element_type=jnp.float32) + b1_ref[...], 0.0)
        z = jnp.maximum(
            jnp.dot(z, w2_ref[...], preferred_element_type=jnp.float32) + b2_ref[...], 0.0)
        logits = jnp.dot(z, w3_ref[...],
                         preferred_element_type=jnp.float32) + b3_ref[...]
        m = jnp.max(logits, axis=1, keepdims=True)
        e = jnp.exp(logits - m)
        lse = jnp.log(jnp.sum(e, axis=1, keepdims=True)) + m
        out_ref[...] = logits - lse

    return pl.pallas_call(
        body,
        out_shape=jax.ShapeDtypeStruct((G, C), jnp.float32),
    )(h, batch2d, fc1W, fc1b, fc2W, fc2b, fc3W, fc3b)


def kernel(x, edge_index, batch, params):
    src = edge_index[0].astype(jnp.int32).reshape(NW, EPT)
    dst = edge_index[1].astype(jnp.int32).reshape(NW, EPT)
    pad = EPAD - EPT
    srcp = jnp.concatenate(
        [src, jnp.zeros((NW, pad), jnp.int32)], axis=1).reshape(NW, CH, 1, K)
    dstp = jnp.concatenate(
        [dst, jnp.full((NW, pad), N, jnp.int32)], axis=1).reshape(NW, CH, 1, K)
    pidx = jnp.concatenate([srcp, dstp], axis=2)
    zeros = jnp.zeros((AGG_ROWS, H), jnp.float32)
    batch2d = batch.astype(jnp.int32).reshape(1, N)

    h = x
    for p in params["convs"]:
        agg2 = _sc_agg(h, pidx, zeros)
        scale = (1.0 + p["eps"]).astype(jnp.float32).reshape(1, 1)
        h = _tc_layer(h, agg2, scale,
                      p["W1"], p["b1"].reshape(1, H),
                      p["W2"], p["b2"].reshape(1, H),
                      p["gamma"].reshape(1, H), p["beta"].reshape(1, H))

    return _tc_head(h, batch2d,
                    params["fc1W"], params["fc1b"].reshape(1, H),
                    params["fc2W"], params["fc2b"].reshape(1, H // 2),
                    params["fc3W"], params["fc3b"].reshape(1, C))
